# baseline (speedup 1.0000x reference)
import functools

import jax
import jax.numpy as jnp
from jax import lax
from jax.experimental import pallas as pl
from jax.experimental.pallas import tpu as pltpu
from jax.experimental.pallas import tpu_sc as plsc

NC = 2
NS = 16
NW = NC * NS

EK = 80
ZR = 128
NP = 10240

BE = 1280
BN = 1024

def _edge_mlp_body(ef_ref, w1, b1, w2, b2, o):
    ef = ef_ref[...]
    h = jnp.dot(ef, w1[...], preferred_element_type=jnp.float32) + b1[...]
    h = jnp.maximum(h, 0.0).astype(jnp.bfloat16)
    g = jnp.dot(h, w2[...], preferred_element_type=jnp.float32) + b2[...]
    o[...] = 1.0 / (1.0 + jnp.exp(-g))


def _edge_mlp(ef, w1, b1, w2, b2, rows=None, off=0):
    E, DE = ef.shape
    rows = E if rows is None else rows
    H = w2.shape[1]
    return pl.pallas_call(
        _edge_mlp_body,
        grid=(rows // BE,),
        in_specs=[pl.BlockSpec((BE, DE), lambda i: (i + off, 0)),
                  pl.BlockSpec(w1.shape, lambda i: (0, 0)),
                  pl.BlockSpec((1, b1.shape[0]), lambda i: (0, 0)),
                  pl.BlockSpec(w2.shape, lambda i: (0, 0)),
                  pl.BlockSpec((1, b2.shape[0]), lambda i: (0, 0))],
        out_specs=pl.BlockSpec((BE, H), lambda i: (i, 0)),
        out_shape=jax.ShapeDtypeStruct((rows, H), jnp.float32),
    )(ef, w1.astype(jnp.bfloat16), b1.reshape(1, -1),
      w2.astype(jnp.bfloat16), b2.reshape(1, -1))


def _support1_body(x_ref, w_ref, b_ref, o_ref):
    o_ref[...] = (jnp.dot(x_ref[...], w_ref[...],
                          preferred_element_type=jnp.float32) + b_ref[...])


def _support1(x, w, b):
    N, D = x.shape
    return pl.pallas_call(
        _support1_body,
        grid=(N // BN,),
        in_specs=[pl.BlockSpec((BN, D), lambda i: (i, 0)),
                  pl.BlockSpec(w.shape, lambda i: (0, 0)),
                  pl.BlockSpec((1, b.shape[0]), lambda i: (0, 0))],
        out_specs=pl.BlockSpec((BN, w.shape[1]), lambda i: (i, 0)),
        out_shape=jax.ShapeDtypeStruct((N, w.shape[1]), jnp.float32),
    )(x, w, b.reshape(1, -1))


def _support2_body(p_ref, w_ref, b_ref, o_ref):
    x = jnp.maximum(p_ref[0] + p_ref[1], 0.0)
    o_ref[...] = (jnp.dot(x, w_ref[...],
                          preferred_element_type=jnp.float32) + b_ref[...])


def _support2(parts, w, b):
    _, N, D = parts.shape
    return pl.pallas_call(
        _support2_body,
        grid=(N // BN,),
        in_specs=[pl.BlockSpec((2, BN, D), lambda i: (0, i, 0)),
                  pl.BlockSpec(w.shape, lambda i: (0, 0)),
                  pl.BlockSpec((1, b.shape[0]), lambda i: (0, 0))],
        out_specs=pl.BlockSpec((BN, w.shape[1]), lambda i: (i, 0)),
        out_shape=jax.ShapeDtypeStruct((N, w.shape[1]), jnp.float32),
    )(parts, w, b.reshape(1, -1))


def _support2b_body(pa_ref, pb_ref, w_ref, b_ref, o_ref):
    x = jnp.maximum(pa_ref[0] + pa_ref[1] + pb_ref[0] + pb_ref[1], 0.0)
    o_ref[...] = (jnp.dot(x, w_ref[...],
                          preferred_element_type=jnp.float32) + b_ref[...])


def _support2b(pa, pb, w, b):
    _, N, D = pa.shape
    return pl.pallas_call(
        _support2b_body,
        grid=(N // BN,),
        in_specs=[pl.BlockSpec((2, BN, D), lambda i: (0, i, 0)),
                  pl.BlockSpec((2, BN, D), lambda i: (0, i, 0)),
                  pl.BlockSpec(w.shape, lambda i: (0, 0)),
                  pl.BlockSpec((1, b.shape[0]), lambda i: (0, 0))],
        out_specs=pl.BlockSpec((BN, w.shape[1]), lambda i: (i, 0)),
        out_shape=jax.ShapeDtypeStruct((N, w.shape[1]), jnp.float32),
    )(pa, pb, w, b.reshape(1, -1))


def _pool_body(nb, p_ref, b_ref, o_ref):
    i = pl.program_id(0)

    @pl.when(i == 0)
    def _init():
        o_ref[...] = jnp.zeros_like(o_ref)

    x = p_ref[0] + p_ref[1]
    seg = b_ref[0, 0, :]
    onehot = (seg[None, :] ==
              lax.broadcasted_iota(jnp.int32, (nb, seg.shape[0]), 0)
              ).astype(jnp.float32)
    o_ref[...] += jnp.dot(onehot, x, preferred_element_type=jnp.float32)


def _pool(parts, batch, nb):
    _, N, D = parts.shape
    return pl.pallas_call(
        functools.partial(_pool_body, nb),
        grid=(N // BN,),
        in_specs=[pl.BlockSpec((2, BN, D), lambda i: (0, i, 0)),
                  pl.BlockSpec((1, 1, BN), lambda i: (i, 0, 0))],
        out_specs=pl.BlockSpec((nb, D), lambda i: (0, 0)),
        out_shape=jax.ShapeDtypeStruct((nb, D), jnp.float32),
        compiler_params=pltpu.CompilerParams(
            dimension_semantics=("arbitrary",)),
    )(parts, batch.reshape(-1, 1, BN))


def _sc_layer_body(ek, ebase, support, ef, esrc, etgt, out,
                   idx_s0, idx_t0, rows0, efb0,
                   idx_s1, idx_t1, rows1, efb1,
                   acc, semg0, seme0, sems0, semg1, seme1, sems1):
    EK = ek
    N = acc.shape[0]
    c = lax.axis_index("c")
    s = lax.axis_index("s")
    wid = c * NS + s
    nrow = N // NS
    eper = ef.shape[0] // NW
    nchunk = eper // EK

    bufs = ((idx_s0, idx_t0, rows0, efb0, semg0, seme0, sems0),
            (idx_s1, idx_t1, rows1, efb1, semg1, seme1, sems1))

    def _zero_rows(j, _):
        for l in range(8):
            sl = pl.ds(l * 16, 16)
            rows0[j, sl] = jnp.zeros((16,), jnp.float32)
        return 0

    lax.fori_loop(0, EK, _zero_rows, 0)
    for r in range(nrow // EK):
        pltpu.sync_copy(rows0, acc.at[pl.ds(s * nrow + r * EK, EK)])
    if nrow % EK:
        pltpu.sync_copy(rows0.at[pl.ds(0, nrow % EK)],
                        acc.at[pl.ds(s * nrow + (nrow // EK) * EK, nrow % EK)])
    plsc.subcore_barrier()

    def _issue(chunk, bb, drain):
        is_, it_, rw, eb, sg, se, ss = bb
        base = wid * eper + chunk * EK
        if drain:
            pltpu.make_async_copy(rw, acc.at[it_], ss).wait()
        pltpu.sync_copy(esrc.at[pl.ds(ebase + base, EK)], is_)
        pltpu.sync_copy(etgt.at[pl.ds(ebase + base, EK)], it_)
        pltpu.async_copy(support.at[is_], rw, sg)
        pltpu.async_copy(ef.at[pl.ds(base, EK)], eb, se)

    def _finish(chunk, bb):
        is_, it_, rw, eb, sg, se, ss = bb
        base = wid * eper + chunk * EK
        pltpu.make_async_copy(support.at[is_], rw, sg).wait()
        pltpu.make_async_copy(ef.at[pl.ds(base, EK)], eb, se).wait()

        def _mul(j, _):
            for m in range(8):
                sl = pl.ds(m * 16, 16)
                rw[j, sl] = rw[j, sl] * eb[j, sl]
            return 0

        lax.fori_loop(0, EK, _mul, 0)
        pltpu.async_copy(rw, acc.at[it_], ss, add=True)

    _issue(0, bufs[0], False)
    _issue(1, bufs[1], False)

    def _pair(i2, _):
        c0 = i2 * 2
        _finish(c0, bufs[0])

        @pl.when(c0 + 2 < nchunk)
        def _prefetch0():
            _issue(c0 + 2, bufs[0], True)

        _finish(c0 + 1, bufs[1])

        @pl.when(c0 + 3 < nchunk)
        def _prefetch1():
            _issue(c0 + 3, bufs[1], True)

        return 0

    lax.fori_loop(0, nchunk // 2, _pair, 0)
    if nchunk % 2:
        _finish(nchunk - 1, bufs[0])
    pltpu.make_async_copy(bufs[0][2], acc.at[bufs[0][1]], bufs[0][6]).wait()
    pltpu.make_async_copy(bufs[1][2], acc.at[bufs[1][1]], bufs[1][6]).wait()
    plsc.subcore_barrier()

    for r in range(nrow // ZR):
        row0 = s * nrow + r * ZR
        pltpu.sync_copy(acc.at[pl.ds(row0, ZR)], out.at[c, pl.ds(row0, ZR)])


def _sc_layer(support, ef, esrc, etgt, ek=EK, ebase=0):
    N, D = support.shape
    mesh = plsc.VectorSubcoreMesh(core_axis_name="c", subcore_axis_name="s",
                                  num_cores=NC, num_subcores=NS)
    return pl.kernel(
        functools.partial(_sc_layer_body, ek, ebase),
        out_type=jax.ShapeDtypeStruct((2, N, D), jnp.float32),
        mesh=mesh,
        scratch_types=[
            pltpu.VMEM((ek,), jnp.int32),
            pltpu.VMEM((ek,), jnp.int32),
            pltpu.VMEM((ek, D), jnp.float32),
            pltpu.VMEM((ek, D), jnp.float32),
            pltpu.VMEM((ek,), jnp.int32),
            pltpu.VMEM((ek,), jnp.int32),
            pltpu.VMEM((ek, D), jnp.float32),
            pltpu.VMEM((ek, D), jnp.float32),
            pltpu.VMEM_SHARED((N, D), jnp.float32),
            pltpu.SemaphoreType.DMA,
            pltpu.SemaphoreType.DMA,
            pltpu.SemaphoreType.DMA,
            pltpu.SemaphoreType.DMA,
            pltpu.SemaphoreType.DMA,
            pltpu.SemaphoreType.DMA,
        ],
    )(support, ef, esrc, etgt)


def kernel(node_features, edge_features, Esrc, Etgt, batch,
           gc1_W, gc1_b, gc2_W, gc2_b, gc3_W, gc3_b,
           ee1_W1, ee1_b1, ee1_W2, ee1_b2,
           ee2_W1, ee2_b1, ee2_W2, ee2_b2,
           ee3_W1, ee3_b1, ee3_W2, ee3_b2):
    esrc = Esrc.astype(jnp.int32)
    etgt = Etgt.astype(jnp.int32)
    nb = 64
    n = node_features.shape[0]
    x = jnp.pad(node_features, ((0, NP - n), (0, 0)))
    batch_p = jnp.pad(batch.astype(jnp.int32), (0, NP - n))
    ef_bf = edge_features.astype(jnp.bfloat16)

    E = esrc.shape[0]
    half_blocks = (E // 2) // BE
    ef1a = _edge_mlp(ef_bf, ee1_W1, ee1_b1, ee1_W2, ee1_b2,
                     rows=E // 2, off=0)
    s1 = _support1(x, gc1_W, gc1_b)
    p1a = _sc_layer(s1, ef1a, esrc, etgt, ek=40, ebase=0)
    ef1b = _edge_mlp(ef_bf, ee1_W1, ee1_b1, ee1_W2, ee1_b2,
                     rows=E // 2, off=half_blocks)
    p1b = _sc_layer(s1, ef1b, esrc, etgt, ek=40, ebase=E // 2)
    ef2 = _edge_mlp(ef_bf, ee2_W1, ee2_b1, ee2_W2, ee2_b2)
    s2 = _support2b(p1a, p1b, gc2_W, gc2_b)
    p2 = _sc_layer(s2, ef2, esrc, etgt)
    ef3 = _edge_mlp(ef_bf, ee3_W1, ee3_b1, ee3_W2, ee3_b2)
    s3 = _support2(p2, gc3_W, gc3_b)
    p3 = _sc_layer(s3, ef3, esrc, etgt)
    return _pool(p3, batch_p, nb)


# speedup vs baseline: 1.0238x; 1.0238x over previous
import functools

import jax
import jax.numpy as jnp
from jax import lax
from jax.experimental import pallas as pl
from jax.experimental.pallas import tpu as pltpu
from jax.experimental.pallas import tpu_sc as plsc

NC = 2
NS = 16
NW = NC * NS

EK = 80
ZR = 128
NP = 10240

BE = 1280
BN = 1024

def _edge_mlp_body(ef_ref, w1, b1, w2, b2, o):
    ef = ef_ref[...]
    h = jnp.dot(ef, w1[...], preferred_element_type=jnp.float32) + b1[...]
    h = jnp.maximum(h, 0.0).astype(jnp.bfloat16)
    g = jnp.dot(h, w2[...], preferred_element_type=jnp.float32) + b2[...]
    o[...] = 1.0 / (1.0 + jnp.exp(-g))


def _edge_mlp(ef, w1, b1, w2, b2):
    E, DE = ef.shape
    H = w2.shape[1]
    return pl.pallas_call(
        _edge_mlp_body,
        grid=(E // BE,),
        in_specs=[pl.BlockSpec((BE, DE), lambda i: (i, 0)),
                  pl.BlockSpec(w1.shape, lambda i: (0, 0)),
                  pl.BlockSpec((1, b1.shape[0]), lambda i: (0, 0)),
                  pl.BlockSpec(w2.shape, lambda i: (0, 0)),
                  pl.BlockSpec((1, b2.shape[0]), lambda i: (0, 0))],
        out_specs=pl.BlockSpec((BE, H), lambda i: (i, 0)),
        out_shape=jax.ShapeDtypeStruct((E, H), jnp.float32),
    )(ef, w1.astype(jnp.bfloat16), b1.reshape(1, -1),
      w2.astype(jnp.bfloat16), b2.reshape(1, -1))


def _support1_body(x_ref, w_ref, b_ref, o_ref):
    o_ref[...] = (jnp.dot(x_ref[...], w_ref[...],
                          preferred_element_type=jnp.float32) + b_ref[...])


def _support1(x, w, b):
    N, D = x.shape
    return pl.pallas_call(
        _support1_body,
        grid=(N // BN,),
        in_specs=[pl.BlockSpec((BN, D), lambda i: (i, 0)),
                  pl.BlockSpec(w.shape, lambda i: (0, 0)),
                  pl.BlockSpec((1, b.shape[0]), lambda i: (0, 0))],
        out_specs=pl.BlockSpec((BN, w.shape[1]), lambda i: (i, 0)),
        out_shape=jax.ShapeDtypeStruct((N, w.shape[1]), jnp.float32),
    )(x, w, b.reshape(1, -1))


def _support2_body(p_ref, w_ref, b_ref, o_ref):
    x = jnp.maximum(p_ref[0] + p_ref[1], 0.0)
    o_ref[...] = (jnp.dot(x, w_ref[...],
                          preferred_element_type=jnp.float32) + b_ref[...])


def _support2(parts, w, b):
    _, N, D = parts.shape
    return pl.pallas_call(
        _support2_body,
        grid=(N // BN,),
        in_specs=[pl.BlockSpec((2, BN, D), lambda i: (0, i, 0)),
                  pl.BlockSpec(w.shape, lambda i: (0, 0)),
                  pl.BlockSpec((1, b.shape[0]), lambda i: (0, 0))],
        out_specs=pl.BlockSpec((BN, w.shape[1]), lambda i: (i, 0)),
        out_shape=jax.ShapeDtypeStruct((N, w.shape[1]), jnp.float32),
    )(parts, w, b.reshape(1, -1))


def _pool_body(nb, p_ref, b_ref, o_ref):
    i = pl.program_id(0)

    @pl.when(i == 0)
    def _init():
        o_ref[...] = jnp.zeros_like(o_ref)

    x = p_ref[0] + p_ref[1]
    seg = b_ref[0, 0, :]
    onehot = (seg[None, :] ==
              lax.broadcasted_iota(jnp.int32, (nb, seg.shape[0]), 0)
              ).astype(jnp.float32)
    o_ref[...] += jnp.dot(onehot, x, preferred_element_type=jnp.float32)


def _pool(parts, batch, nb):
    _, N, D = parts.shape
    return pl.pallas_call(
        functools.partial(_pool_body, nb),
        grid=(N // BN,),
        in_specs=[pl.BlockSpec((2, BN, D), lambda i: (0, i, 0)),
                  pl.BlockSpec((1, 1, BN), lambda i: (i, 0, 0))],
        out_specs=pl.BlockSpec((nb, D), lambda i: (0, 0)),
        out_shape=jax.ShapeDtypeStruct((nb, D), jnp.float32),
        compiler_params=pltpu.CompilerParams(
            dimension_semantics=("arbitrary",)),
    )(parts, batch.reshape(-1, 1, BN))


def _sc_layer_body(support, ef, esrc, etgt, out,
                   idx_s0, idx_t0, rows0, efb0,
                   idx_s1, idx_t1, rows1, efb1,
                   acc, semg0, seme0, sems0, semg1, seme1, sems1):
    N = acc.shape[0]
    E = esrc.shape[0]
    c = lax.axis_index("c")
    s = lax.axis_index("s")
    wid = c * NS + s
    nrow = N // NS
    eper = E // NW
    nchunk = eper // EK

    bufs = ((idx_s0, idx_t0, rows0, efb0, semg0, seme0, sems0),
            (idx_s1, idx_t1, rows1, efb1, semg1, seme1, sems1))

    def _zero_rows(j, _):
        for l in range(8):
            sl = pl.ds(l * 16, 16)
            rows0[j, sl] = jnp.zeros((16,), jnp.float32)
        return 0

    lax.fori_loop(0, EK, _zero_rows, 0)
    for r in range(nrow // EK):
        pltpu.sync_copy(rows0, acc.at[pl.ds(s * nrow + r * EK, EK)])
    plsc.subcore_barrier()

    def _issue(chunk, bb, drain):
        is_, it_, rw, eb, sg, se, ss = bb
        base = wid * eper + chunk * EK
        if drain:
            pltpu.make_async_copy(rw, acc.at[it_], ss).wait()
        pltpu.sync_copy(esrc.at[pl.ds(base, EK)], is_)
        pltpu.sync_copy(etgt.at[pl.ds(base, EK)], it_)
        pltpu.async_copy(support.at[is_], rw, sg)
        pltpu.async_copy(ef.at[pl.ds(base, EK)], eb, se)

    def _finish(chunk, bb):
        is_, it_, rw, eb, sg, se, ss = bb
        base = wid * eper + chunk * EK
        pltpu.make_async_copy(support.at[is_], rw, sg).wait()
        pltpu.make_async_copy(ef.at[pl.ds(base, EK)], eb, se).wait()

        def _mul(j, _):
            for m in range(8):
                sl = pl.ds(m * 16, 16)
                rw[j, sl] = rw[j, sl] * eb[j, sl]
            return 0

        lax.fori_loop(0, EK, _mul, 0)
        pltpu.async_copy(rw, acc.at[it_], ss, add=True)

    _issue(0, bufs[0], False)
    _issue(1, bufs[1], False)

    def _pair(i2, _):
        c0 = i2 * 2
        _finish(c0, bufs[0])

        @pl.when(c0 + 2 < nchunk)
        def _prefetch0():
            _issue(c0 + 2, bufs[0], True)

        _finish(c0 + 1, bufs[1])

        @pl.when(c0 + 3 < nchunk)
        def _prefetch1():
            _issue(c0 + 3, bufs[1], True)

        return 0

    lax.fori_loop(0, nchunk // 2, _pair, 0)
    if nchunk % 2:
        _finish(nchunk - 1, bufs[0])
    pltpu.make_async_copy(bufs[0][2], acc.at[bufs[0][1]], bufs[0][6]).wait()
    pltpu.make_async_copy(bufs[1][2], acc.at[bufs[1][1]], bufs[1][6]).wait()
    plsc.subcore_barrier()

    for r in range(nrow // ZR):
        row0 = s * nrow + r * ZR
        pltpu.sync_copy(acc.at[pl.ds(row0, ZR)], out.at[c, pl.ds(row0, ZR)])


def _sc_layer(support, ef, esrc, etgt):
    N, D = support.shape
    mesh = plsc.VectorSubcoreMesh(core_axis_name="c", subcore_axis_name="s",
                                  num_cores=NC, num_subcores=NS)
    return pl.kernel(
        _sc_layer_body,
        out_type=jax.ShapeDtypeStruct((2, N, D), jnp.float32),
        mesh=mesh,
        scratch_types=[
            pltpu.VMEM((EK,), jnp.int32),
            pltpu.VMEM((EK,), jnp.int32),
            pltpu.VMEM((EK, D), jnp.float32),
            pltpu.VMEM((EK, D), jnp.float32),
            pltpu.VMEM((EK,), jnp.int32),
            pltpu.VMEM((EK,), jnp.int32),
            pltpu.VMEM((EK, D), jnp.float32),
            pltpu.VMEM((EK, D), jnp.float32),
            pltpu.VMEM_SHARED((N, D), jnp.float32),
            pltpu.SemaphoreType.DMA,
            pltpu.SemaphoreType.DMA,
            pltpu.SemaphoreType.DMA,
            pltpu.SemaphoreType.DMA,
            pltpu.SemaphoreType.DMA,
            pltpu.SemaphoreType.DMA,
        ],
    )(support, ef, esrc, etgt)


def kernel(node_features, edge_features, Esrc, Etgt, batch,
           gc1_W, gc1_b, gc2_W, gc2_b, gc3_W, gc3_b,
           ee1_W1, ee1_b1, ee1_W2, ee1_b2,
           ee2_W1, ee2_b1, ee2_W2, ee2_b2,
           ee3_W1, ee3_b1, ee3_W2, ee3_b2):
    esrc = Esrc.astype(jnp.int32)
    etgt = Etgt.astype(jnp.int32)
    nb = 64
    n = node_features.shape[0]
    x = jnp.pad(node_features, ((0, NP - n), (0, 0)))
    batch_p = jnp.pad(batch.astype(jnp.int32), (0, NP - n))
    ef_bf = edge_features.astype(jnp.bfloat16)

    ef1 = _edge_mlp(ef_bf, ee1_W1, ee1_b1, ee1_W2, ee1_b2)
    s1 = _support1(x, gc1_W, gc1_b)
    p1 = _sc_layer(s1, ef1, esrc, etgt)
    ef2 = _edge_mlp(ef_bf, ee2_W1, ee2_b1, ee2_W2, ee2_b2)
    s2 = _support2(p1, gc2_W, gc2_b)
    p2 = _sc_layer(s2, ef2, esrc, etgt)
    ef3 = _edge_mlp(ef_bf, ee3_W1, ee3_b1, ee3_W2, ee3_b2)
    s3 = _support2(p2, gc3_W, gc3_b)
    p3 = _sc_layer(s3, ef3, esrc, etgt)
    return _pool(p3, batch_p, nb)
